# baseline (speedup 1.0000x reference)
import functools

import jax
import jax.numpy as jnp
from jax import lax
from jax.experimental import pallas as pl
from jax.experimental.pallas import tpu as pltpu
from jax.experimental.pallas import tpu_sc as plsc

_LANES = 128
_NBUF = 4


def _sc_gather(table128, idx_flat, nch, ch):
    n_total = idx_flat.shape[0]
    n_per_w = nch * ch
    mesh = plsc.VectorSubcoreMesh(core_axis_name="c", subcore_axis_name="s")
    info = plsc.get_sparse_core_info()
    nc = info.num_cores

    @functools.partial(
        pl.kernel,
        mesh=mesh,
        out_type=jax.ShapeDtypeStruct((n_total, _LANES), jnp.float32),
        scratch_types=[
            pltpu.VMEM((n_per_w,), jnp.int32),
            pltpu.VMEM((_NBUF, ch, _LANES), jnp.float32),
        ]
        + [pltpu.SemaphoreType.DMA] * (2 * _NBUF),
    )
    def k(table_hbm, idx_hbm, out_hbm, idx_v, rows_v, *sems):
        gsem, wsem = sems[:_NBUF], sems[_NBUF:]
        wid = lax.axis_index("s") * nc + lax.axis_index("c")
        base = wid * n_per_w
        pltpu.sync_copy(idx_hbm.at[pl.ds(base, n_per_w)], idx_v)
        gcp = [None] * _NBUF
        wcp = [None] * _NBUF

        def start_gather(j):
            b = j % _NBUF
            if wcp[b] is not None:
                wcp[b].wait()
            gcp[b] = pltpu.async_copy(
                table_hbm.at[idx_v.at[pl.ds(j * ch, ch)]],
                rows_v.at[b],
                gsem[b],
            )

        def start_writeback(j):
            b = j % _NBUF
            gcp[b].wait()
            wcp[b] = pltpu.async_copy(
                rows_v.at[b],
                out_hbm.at[pl.ds(base + j * ch, ch)],
                wsem[b],
            )

        depth = _NBUF - 1
        for j in range(nch):
            start_gather(j)
            if j >= depth:
                start_writeback(j - depth)
        for j in range(max(nch - depth, 0), nch):
            start_writeback(j)
        for b in range(_NBUF):
            if wcp[b] is not None:
                wcp[b].wait()

    return k(table128, idx_flat)


def _attn_body(emb_ref, pos_ref, wk_ref, wq_ref, wv_ref, wl_ref, bl_ref,
               mask_ref, out_ref, *, scale, bb, tp, t_out):
    e = emb_ref[...] + pos_ref[...]
    q = jnp.dot(e, wq_ref[...], preferred_element_type=jnp.float32)
    k = jnp.dot(e, wk_ref[...], preferred_element_type=jnp.float32)
    v = jnp.dot(e, wv_ref[...], preferred_element_type=jnp.float32)
    wei = lax.dot_general(q, k, (((1,), (1,)), ((), ())),
                          preferred_element_type=jnp.float32)
    wei = wei * scale + mask_ref[...]
    m = jnp.max(wei, axis=1, keepdims=True)
    p = jnp.exp(wei - m)
    s = jnp.sum(p, axis=1, keepdims=True)
    o = jnp.dot(p, v, preferred_element_type=jnp.float32) / s
    logits = jnp.dot(o, wl_ref[...],
                     preferred_element_type=jnp.float32) + bl_ref[...]
    vv = logits.shape[1]
    for b in range(bb):
        out_ref[b] = lax.slice(logits, (b * tp, 0), (b * tp + t_out, vv))


def _tc_attn_logits(emb2d, pos_tiled, Wk, Wq, Wv, Wl, bl2d, mask_add,
                    bb, tp, t_out, n_batch):
    D = Wl.shape[0]
    V = Wl.shape[1]
    R = bb * tp
    grid = n_batch // bb
    scale = float(D) ** -0.5
    return pl.pallas_call(
        functools.partial(_attn_body, scale=scale, bb=bb, tp=tp, t_out=t_out),
        grid=(grid,),
        in_specs=[
            pl.BlockSpec((R, _LANES), lambda i: (i, 0)),
            pl.BlockSpec((R, _LANES), lambda i: (0, 0)),
            pl.BlockSpec((_LANES, D), lambda i: (0, 0)),
            pl.BlockSpec((_LANES, D), lambda i: (0, 0)),
            pl.BlockSpec((_LANES, D), lambda i: (0, 0)),
            pl.BlockSpec((D, V), lambda i: (0, 0)),
            pl.BlockSpec((1, V), lambda i: (0, 0)),
            pl.BlockSpec((R, R), lambda i: (0, 0)),
        ],
        out_specs=pl.BlockSpec((bb, t_out, V), lambda i: (i, 0, 0)),
        out_shape=jax.ShapeDtypeStruct((n_batch, t_out, V), jnp.float32),
        compiler_params=pltpu.CompilerParams(
            dimension_semantics=("parallel",),
        ),
    )(emb2d, pos_tiled, Wk, Wq, Wv, Wl, bl2d, mask_add)


def kernel(x, tok_table, pos_table, Wk, Wq, Wv, Wl, bl):
    B, T = x.shape
    V, D = tok_table.shape
    TP = 56
    N = B * TP

    BB = 8
    R = BB * TP

    info = plsc.get_sparse_core_info()
    NW = info.num_cores * info.num_subcores
    n_per_w = N // NW
    CH = 128
    NCH = n_per_w // CH
    tok128 = jnp.pad(tok_table, ((0, 0), (0, _LANES - D)))
    idx_flat = jnp.pad(x.astype(jnp.int32), ((0, 0), (0, TP - T))).reshape(N)
    emb2d = _sc_gather(tok128, idx_flat, NCH, CH)

    pos128 = jnp.pad(pos_table, ((0, TP - T), (0, _LANES - D)))
    pos_tiled = jnp.tile(pos128, (BB, 1))
    wpad = ((0, _LANES - D), (0, 0))
    Wk128, Wq128, Wv128 = (jnp.pad(W, wpad) for W in (Wk, Wq, Wv))
    r = jnp.arange(R)
    bidx, t = r // TP, r % TP
    causal = ((bidx[:, None] == bidx[None, :])
              & (t[:, None] >= t[None, :])
              & (t[None, :] < T))
    mask_add = jnp.where(causal, 0.0, -1e30).astype(jnp.float32)
    return _tc_attn_logits(emb2d, pos_tiled, Wk128, Wq128, Wv128, Wl,
                           bl.reshape(1, V), mask_add, BB, TP, T, B)


# speedup vs baseline: 1.4353x; 1.4353x over previous
import functools

import jax
import jax.numpy as jnp
from jax import lax
from jax.experimental import pallas as pl
from jax.experimental.pallas import tpu as pltpu
from jax.experimental.pallas import tpu_sc as plsc

_PACK = 4


def _sc_gather(table, idx_flat, nch, ch):
    n_total = idx_flat.shape[0]
    d = table.shape[1]
    n_per_w = nch * ch
    mesh = plsc.VectorSubcoreMesh(core_axis_name="c", subcore_axis_name="s")
    info = plsc.get_sparse_core_info()
    nc = info.num_cores

    @functools.partial(
        pl.kernel,
        mesh=mesh,
        out_type=jax.ShapeDtypeStruct((n_total, d), jnp.float32),
        scratch_types=[
            pltpu.VMEM((n_per_w,), jnp.int32),
            pltpu.VMEM((2, ch, d), jnp.float32),
            pltpu.SemaphoreType.DMA,
            pltpu.SemaphoreType.DMA,
        ],
        compiler_params=pltpu.CompilerParams(use_tc_tiling_on_sc=False),
    )
    def k(table_hbm, idx_hbm, out_hbm, idx_v, rows_v, sem0, sem1):
        wid = lax.axis_index("s") * nc + lax.axis_index("c")
        base = wid * n_per_w
        pltpu.sync_copy(idx_hbm.at[pl.ds(base, n_per_w)], idx_v)
        sems = (sem0, sem1)
        cps = [None, None]
        for j in range(nch):
            b = j % 2
            cps[b] = pltpu.async_copy(
                table_hbm.at[idx_v.at[pl.ds(j * ch, ch)]],
                rows_v.at[b],
                sems[b],
            )
            if j >= 1:
                bp = (j - 1) % 2
                cps[bp].wait()
                pltpu.sync_copy(
                    rows_v.at[bp],
                    out_hbm.at[pl.ds(base + (j - 1) * ch, ch)],
                )
        bl_ = (nch - 1) % 2
        cps[bl_].wait()
        pltpu.sync_copy(
            rows_v.at[bl_],
            out_hbm.at[pl.ds(base + (nch - 1) * ch, ch)],
        )

    return k(table, idx_flat)


def _attn_body(emb_ref, pos_ref, wk_ref, wq_ref, wv_ref, wl_ref, bl_ref,
               mask_ref, out_ref, *, scale, bb, tp, t_out, d):
    e4 = emb_ref[...] + pos_ref[...]
    rp = e4.shape[0]
    e = jnp.concatenate(
        [lax.slice(e4, (0, j * d), (rp, (j + 1) * d)) for j in range(_PACK)],
        axis=0,
    )
    q = jnp.dot(e, wq_ref[...], preferred_element_type=jnp.float32)
    k = jnp.dot(e, wk_ref[...], preferred_element_type=jnp.float32)
    v = jnp.dot(e, wv_ref[...], preferred_element_type=jnp.float32)
    wei = lax.dot_general(q, k, (((1,), (1,)), ((), ())),
                          preferred_element_type=jnp.float32)
    wei = wei * scale + mask_ref[...]
    m = jnp.max(wei, axis=1, keepdims=True)
    p = jnp.exp(wei - m)
    s = jnp.sum(p, axis=1, keepdims=True)
    o = jnp.dot(p, v, preferred_element_type=jnp.float32) / s
    logits = jnp.dot(o, wl_ref[...],
                     preferred_element_type=jnp.float32) + bl_ref[...]
    vv = logits.shape[1]
    for b in range(bb):
        out_ref[b] = lax.slice(logits, (b * tp, 0), (b * tp + t_out, vv))


def _tc_attn_logits(emb_packed, pos_packed, Wk, Wq, Wv, Wl, bl2d, mask_add,
                    bb, tp, t_out, n_batch):
    D = Wl.shape[0]
    V = Wl.shape[1]
    R = bb * tp
    RP = R // _PACK
    grid = n_batch // bb
    scale = float(D) ** -0.5
    return pl.pallas_call(
        functools.partial(_attn_body, scale=scale, bb=bb, tp=tp,
                          t_out=t_out, d=D),
        grid=(grid,),
        in_specs=[
            pl.BlockSpec((RP, _PACK * D), lambda i: (i, 0)),
            pl.BlockSpec((RP, _PACK * D), lambda i: (0, 0)),
            pl.BlockSpec((D, D), lambda i: (0, 0)),
            pl.BlockSpec((D, D), lambda i: (0, 0)),
            pl.BlockSpec((D, D), lambda i: (0, 0)),
            pl.BlockSpec((D, V), lambda i: (0, 0)),
            pl.BlockSpec((1, V), lambda i: (0, 0)),
            pl.BlockSpec((R, R), lambda i: (0, 0)),
        ],
        out_specs=pl.BlockSpec((bb, t_out, V), lambda i: (i, 0, 0)),
        out_shape=jax.ShapeDtypeStruct((n_batch, t_out, V), jnp.float32),
        compiler_params=pltpu.CompilerParams(
            dimension_semantics=("parallel",),
        ),
    )(emb_packed, pos_packed, Wk, Wq, Wv, Wl, bl2d, mask_add)


def kernel(x, tok_table, pos_table, Wk, Wq, Wv, Wl, bl):
    B, T = x.shape
    V, D = tok_table.shape
    TP = 56
    N = B * TP

    BB = 8
    R = BB * TP
    RP = R // _PACK

    info = plsc.get_sparse_core_info()
    NW = info.num_cores * info.num_subcores
    n_per_w = N // NW
    CH = 112
    NCH = n_per_w // CH
    x_pad = jnp.pad(x.astype(jnp.int32), ((0, 0), (0, TP - T))).reshape(N)
    m = jnp.arange(N, dtype=jnp.int32)
    g, i_loc, j = m // R, (m % R) // _PACK, m % _PACK
    idx_perm = x_pad[g * R + j * RP + i_loc]
    emb = _sc_gather(tok_table, idx_perm, NCH, CH)
    emb_packed = emb.reshape(N // _PACK, _PACK * D)

    pos_pad = jnp.pad(pos_table, ((0, TP - T), (0, 0)))
    pos_packed = jnp.tile(pos_pad[jnp.arange(RP) % TP], (1, _PACK))
    r = jnp.arange(R)
    bidx, t = r // TP, r % TP
    causal = ((bidx[:, None] == bidx[None, :])
              & (t[:, None] >= t[None, :])
              & (t[None, :] < T))
    mask_add = jnp.where(causal, 0.0, -1e30).astype(jnp.float32)
    return _tc_attn_logits(emb_packed, pos_packed, Wk, Wq, Wv, Wl,
                           bl.reshape(1, V), mask_add, BB, TP, T, B)


# speedup vs baseline: 1.4462x; 1.0076x over previous
import functools

import jax
import jax.numpy as jnp
from jax import lax
from jax.experimental import pallas as pl
from jax.experimental.pallas import tpu as pltpu
from jax.experimental.pallas import tpu_sc as plsc

_PACK = 4


def _sc_gather(table, idx_flat, nch, ch):
    n_total = idx_flat.shape[0]
    d = table.shape[1]
    n_per_w = nch * ch
    w_words = n_per_w * d
    mesh = plsc.VectorSubcoreMesh(core_axis_name="c", subcore_axis_name="s")
    info = plsc.get_sparse_core_info()
    nc = info.num_cores

    @functools.partial(
        pl.kernel,
        mesh=mesh,
        out_type=jax.ShapeDtypeStruct((n_total * d,), jnp.float32),
        scratch_types=[
            pltpu.VMEM((n_per_w,), jnp.int32),
            pltpu.VMEM((n_per_w, d), jnp.float32),
            pltpu.VMEM((w_words,), jnp.float32),
            pltpu.SemaphoreType.DMA,
        ],
        compiler_params=pltpu.CompilerParams(use_tc_tiling_on_sc=False),
    )
    def k(table_hbm, idx_hbm, out_hbm, idx_v, rows_v, flat_v, sem):
        wid = lax.axis_index("s") * nc + lax.axis_index("c")
        base = wid * n_per_w
        pltpu.sync_copy(idx_hbm.at[pl.ds(base, n_per_w)], idx_v)
        cps = []
        for j in range(nch):
            cps.append(pltpu.async_copy(
                table_hbm.at[idx_v.at[pl.ds(j * ch, ch)]],
                rows_v.at[pl.ds(j * ch, ch)],
                sem,
            ))
        for c in cps:
            c.wait()

        nvec = d // 16

        def repack(g, _):
            row = g * 4
            woff = g * 4 * d
            for q in range(4):
                for h in range(nvec):
                    flat_v[pl.ds(woff + q * d + h * 16, 16)] = (
                        rows_v[row + q, pl.ds(h * 16, 16)]
                    )
            return 0

        lax.fori_loop(0, n_per_w // 4, repack, 0)
        pltpu.sync_copy(flat_v, out_hbm.at[pl.ds(wid * w_words, w_words)])

    return k(table, idx_flat)


def _attn_body(emb_ref, pos_ref, wk_ref, wq_ref, wv_ref, wl_ref, bl_ref,
               mask_ref, out_ref, *, scale, bb, tp, t_out, d):
    e4 = emb_ref[...] + pos_ref[...]
    rp = e4.shape[0]
    e = jnp.concatenate(
        [lax.slice(e4, (0, j * d), (rp, (j + 1) * d)) for j in range(_PACK)],
        axis=0,
    )
    q = jnp.dot(e, wq_ref[...], preferred_element_type=jnp.float32)
    k = jnp.dot(e, wk_ref[...], preferred_element_type=jnp.float32)
    v = jnp.dot(e, wv_ref[...], preferred_element_type=jnp.float32)
    wei = lax.dot_general(q, k, (((1,), (1,)), ((), ())),
                          preferred_element_type=jnp.float32)
    wei = wei * scale + mask_ref[...]
    m = jnp.max(wei, axis=1, keepdims=True)
    p = jnp.exp(wei - m)
    s = jnp.sum(p, axis=1, keepdims=True)
    o = jnp.dot(p, v, preferred_element_type=jnp.float32) / s
    logits = jnp.dot(o, wl_ref[...],
                     preferred_element_type=jnp.float32) + bl_ref[...]
    vv = logits.shape[1]
    for b in range(bb):
        out_ref[b] = lax.slice(logits, (b * tp, 0), (b * tp + t_out, vv))


def _tc_attn_logits(emb_packed, pos_packed, Wk, Wq, Wv, Wl, bl2d, mask_add,
                    bb, tp, t_out, n_batch):
    D = Wl.shape[0]
    V = Wl.shape[1]
    R = bb * tp
    RP = R // _PACK
    grid = n_batch // bb
    scale = float(D) ** -0.5
    return pl.pallas_call(
        functools.partial(_attn_body, scale=scale, bb=bb, tp=tp,
                          t_out=t_out, d=D),
        grid=(grid,),
        in_specs=[
            pl.BlockSpec((RP, _PACK * D), lambda i: (i, 0)),
            pl.BlockSpec((RP, _PACK * D), lambda i: (0, 0)),
            pl.BlockSpec((D, D), lambda i: (0, 0)),
            pl.BlockSpec((D, D), lambda i: (0, 0)),
            pl.BlockSpec((D, D), lambda i: (0, 0)),
            pl.BlockSpec((D, V), lambda i: (0, 0)),
            pl.BlockSpec((1, V), lambda i: (0, 0)),
            pl.BlockSpec((R, R), lambda i: (0, 0)),
        ],
        out_specs=pl.BlockSpec((bb, t_out, V), lambda i: (i, 0, 0)),
        out_shape=jax.ShapeDtypeStruct((n_batch, t_out, V), jnp.float32),
        compiler_params=pltpu.CompilerParams(
            dimension_semantics=("parallel",),
        ),
    )(emb_packed, pos_packed, Wk, Wq, Wv, Wl, bl2d, mask_add)


def kernel(x, tok_table, pos_table, Wk, Wq, Wv, Wl, bl):
    B, T = x.shape
    V, D = tok_table.shape
    TP = 56
    N = B * TP

    BB = 8
    R = BB * TP
    RP = R // _PACK

    info = plsc.get_sparse_core_info()
    NW = info.num_cores * info.num_subcores
    n_per_w = N // NW
    CH = 112
    NCH = n_per_w // CH
    x_pad = jnp.pad(x.astype(jnp.int32), ((0, 0), (0, TP - T))).reshape(N)
    m = jnp.arange(N, dtype=jnp.int32)
    g, i_loc, j = m // R, (m % R) // _PACK, m % _PACK
    idx_perm = x_pad[g * R + j * RP + i_loc]
    emb = _sc_gather(tok_table, idx_perm, NCH, CH)
    emb_packed = emb.reshape(N // _PACK, _PACK * D)

    pos_pad = jnp.pad(pos_table, ((0, TP - T), (0, 0)))
    pos_packed = jnp.tile(pos_pad[jnp.arange(RP) % TP], (1, _PACK))
    r = jnp.arange(R)
    bidx, t = r // TP, r % TP
    causal = ((bidx[:, None] == bidx[None, :])
              & (t[:, None] >= t[None, :])
              & (t[None, :] < T))
    mask_add = jnp.where(causal, 0.0, -1e30).astype(jnp.float32)
    return _tc_attn_logits(emb_packed, pos_packed, Wk, Wq, Wv, Wl,
                           bl.reshape(1, V), mask_add, BB, TP, T, B)


# speedup vs baseline: 2.0915x; 1.4462x over previous
import functools

import jax
import jax.numpy as jnp
from jax import lax
from jax.experimental import pallas as pl
from jax.experimental.pallas import tpu as pltpu
from jax.experimental.pallas import tpu_sc as plsc

_PACK = 4
_TCH = 8
_BCH = 128


def _sc_gather(table, idx_flat, nch, ch):
    n_total = idx_flat.shape[0]
    d = table.shape[1]
    n_per_w = nch * ch
    w_words = n_per_w * d
    mesh = plsc.VectorSubcoreMesh(core_axis_name="c", subcore_axis_name="s")
    info = plsc.get_sparse_core_info()
    nc = info.num_cores

    @functools.partial(
        pl.kernel,
        mesh=mesh,
        out_type=jax.ShapeDtypeStruct((n_total * d,), jnp.float32),
        scratch_types=[
            pltpu.VMEM((n_per_w,), jnp.int32),
            pltpu.VMEM((n_per_w, d), jnp.float32),
            pltpu.VMEM((w_words,), jnp.float32),
            pltpu.SemaphoreType.DMA,
        ],
        compiler_params=pltpu.CompilerParams(use_tc_tiling_on_sc=False),
    )
    def k(table_hbm, idx_hbm, out_hbm, idx_v, rows_v, flat_v, sem):
        wid = lax.axis_index("s") * nc + lax.axis_index("c")
        base = wid * n_per_w
        pltpu.sync_copy(idx_hbm.at[pl.ds(base, n_per_w)], idx_v)
        cps = []
        for j in range(nch):
            cps.append(pltpu.async_copy(
                table_hbm.at[idx_v.at[pl.ds(j * ch, ch)]],
                rows_v.at[pl.ds(j * ch, ch)],
                sem,
            ))
        for c in cps:
            c.wait()

        nvec = d // 16

        def repack(g, _):
            row = g * 4
            woff = g * 4 * d
            for q in range(4):
                for h in range(nvec):
                    flat_v[pl.ds(woff + q * d + h * 16, 16)] = (
                        rows_v[row + q, pl.ds(h * 16, 16)]
                    )
            return 0

        lax.fori_loop(0, n_per_w // 4, repack, 0)
        pltpu.sync_copy(flat_v, out_hbm.at[pl.ds(wid * w_words, w_words)])

    return k(table, idx_flat)


def _attn_body(emb_ref, pos_ref, wk_ref, wq_ref, wv_ref, mask_ref, out_ref,
               *, scale, tp, d):
    e4 = emb_ref[...] + pos_ref[...]
    rp = e4.shape[0]
    e = jnp.concatenate(
        [lax.slice(e4, (0, j * d), (rp, (j + 1) * d)) for j in range(_PACK)],
        axis=0,
    )
    q = jnp.dot(e, wq_ref[...], preferred_element_type=jnp.float32)
    k = jnp.dot(e, wk_ref[...], preferred_element_type=jnp.float32)
    v = jnp.dot(e, wv_ref[...], preferred_element_type=jnp.float32)
    wei = lax.dot_general(q, k, (((1,), (1,)), ((), ())),
                          preferred_element_type=jnp.float32)
    wei = wei * scale + mask_ref[...]
    m = jnp.max(wei, axis=1, keepdims=True)
    p = jnp.exp(wei - m)
    s = jnp.sum(p, axis=1, keepdims=True)
    o = jnp.dot(p, v, preferred_element_type=jnp.float32) / s
    halves = []
    for rq in range(2):
        halves.append(jnp.concatenate(
            [lax.slice(o, ((2 * j + rq) * tp, 0), ((2 * j + rq + 1) * tp, d))
             for j in range(_PACK)],
            axis=1,
        ))
    out_ref[...] = jnp.concatenate(halves, axis=0)


def _tc_attn(emb_packed, pos_packed, Wk, Wq, Wv, mask_add, bb, tp, n_batch):
    D = Wk.shape[0]
    R = bb * tp
    RP = R // _PACK
    grid = n_batch // bb
    scale = float(D) ** -0.5
    return pl.pallas_call(
        functools.partial(_attn_body, scale=scale, tp=tp, d=D),
        grid=(grid,),
        in_specs=[
            pl.BlockSpec((RP, _PACK * D), lambda i: (i, 0)),
            pl.BlockSpec((RP, _PACK * D), lambda i: (0, 0)),
            pl.BlockSpec((D, D), lambda i: (0, 0)),
            pl.BlockSpec((D, D), lambda i: (0, 0)),
            pl.BlockSpec((D, D), lambda i: (0, 0)),
            pl.BlockSpec((R, R), lambda i: (0, 0)),
        ],
        out_specs=pl.BlockSpec((2 * tp, _PACK * D), lambda i: (i, 0)),
        out_shape=jax.ShapeDtypeStruct((n_batch // _PACK * tp, _PACK * D),
                                       jnp.float32),
        compiler_params=pltpu.CompilerParams(
            dimension_semantics=("parallel",),
        ),
    )(emb_packed, pos_packed, Wk, Wq, Wv, mask_add)


def _proj_body(o_ref, wlt_ref, bl_ref, out_ref, *, d):
    nb4 = o_ref.shape[0]
    for tl in range(_TCH):
        o4 = o_ref[:, tl, :]
        ot = jnp.concatenate(
            [lax.slice(o4, (0, j * d), (nb4, (j + 1) * d))
             for j in range(_PACK)],
            axis=0,
        )
        lg = lax.dot_general(wlt_ref[...], ot, (((1,), (1,)), ((), ())),
                             preferred_element_type=jnp.float32)
        out_ref[tl] = lg + bl_ref[...]


def _tc_proj(o3d, WlT, bl_col, t_out, tp):
    V, D = WlT.shape
    nb4 = o3d.shape[0]
    n_batch = nb4 * _PACK
    nbc = _BCH // _PACK
    grid = (pl.cdiv(t_out, _TCH), n_batch // _BCH)
    return pl.pallas_call(
        functools.partial(_proj_body, d=D),
        grid=grid,
        in_specs=[
            pl.BlockSpec((nbc, _TCH, _PACK * D), lambda u, bc: (bc, u, 0)),
            pl.BlockSpec((V, D), lambda u, bc: (0, 0)),
            pl.BlockSpec((V, 1), lambda u, bc: (0, 0)),
        ],
        out_specs=pl.BlockSpec((_TCH, V, _BCH), lambda u, bc: (u, 0, bc)),
        out_shape=jax.ShapeDtypeStruct((t_out, V, n_batch), jnp.float32),
        compiler_params=pltpu.CompilerParams(
            dimension_semantics=("parallel", "parallel"),
        ),
    )(o3d, WlT, bl_col)


def kernel(x, tok_table, pos_table, Wk, Wq, Wv, Wl, bl):
    B, T = x.shape
    V, D = tok_table.shape
    TP = 56
    N = B * TP

    BB = 8
    R = BB * TP
    RP = R // _PACK

    info = plsc.get_sparse_core_info()
    NW = info.num_cores * info.num_subcores
    n_per_w = N // NW
    CH = 112
    NCH = n_per_w // CH
    x_pad = jnp.pad(x.astype(jnp.int32), ((0, 0), (0, TP - T))).reshape(N)
    m = jnp.arange(N, dtype=jnp.int32)
    gi, mm = m // R, m % R
    tau = (mm % _PACK) * RP + mm // _PACK
    lv, tt = tau // TP, tau % TP
    b4 = 2 * gi + (lv % 2)
    nbc = _BCH // _PACK
    breal = (b4 // nbc) * _BCH + (lv // 2) * nbc + (b4 % nbc)
    idx_perm = x_pad[breal * TP + tt]
    emb_flat = _sc_gather(tok_table, idx_perm, NCH, CH)
    emb_packed = emb_flat.reshape(N // _PACK, _PACK * D)

    pos_pad = jnp.pad(pos_table, ((0, TP - T), (0, 0)))
    pos_packed = jnp.tile(pos_pad[jnp.arange(RP) % TP], (1, _PACK))
    r = jnp.arange(R)
    bidx, t = r // TP, r % TP
    causal = ((bidx[:, None] == bidx[None, :])
              & (t[:, None] >= t[None, :])
              & (t[None, :] < T))
    mask_add = jnp.where(causal, 0.0, -1e30).astype(jnp.float32)
    o2d = _tc_attn(emb_packed, pos_packed, Wk, Wq, Wv, mask_add, BB, TP, B)
    o3d = o2d.reshape(B // _PACK, TP, _PACK * D)

    out_tvb = _tc_proj(o3d, Wl.T, bl.reshape(V, 1), T, TP)
    return jnp.transpose(out_tvb, (2, 0, 1))


# speedup vs baseline: 2.1256x; 1.0163x over previous
import functools

import jax
import jax.numpy as jnp
from jax import lax
from jax.experimental import pallas as pl
from jax.experimental.pallas import tpu as pltpu
from jax.experimental.pallas import tpu_sc as plsc

_PACK = 4
_TCH = 8
_BCH = 128


def _sc_gather(table, idx_flat, nch, ch):
    n_total = idx_flat.shape[0]
    d = table.shape[1]
    n_per_w = nch * ch
    w_words = n_per_w * d
    mesh = plsc.VectorSubcoreMesh(core_axis_name="c", subcore_axis_name="s")
    info = plsc.get_sparse_core_info()
    nc = info.num_cores

    @functools.partial(
        pl.kernel,
        mesh=mesh,
        out_type=jax.ShapeDtypeStruct((n_total * d,), jnp.float32),
        scratch_types=[
            pltpu.VMEM((n_per_w,), jnp.int32),
            pltpu.VMEM((n_per_w, d), jnp.float32),
            pltpu.VMEM((w_words,), jnp.float32),
            pltpu.SemaphoreType.DMA,
        ],
        compiler_params=pltpu.CompilerParams(use_tc_tiling_on_sc=False),
    )
    def k(table_hbm, idx_hbm, out_hbm, idx_v, rows_v, flat_v, sem):
        wid = lax.axis_index("s") * nc + lax.axis_index("c")
        base = wid * n_per_w
        pltpu.sync_copy(idx_hbm.at[pl.ds(base, n_per_w)], idx_v)
        cps = []
        for j in range(nch):
            cps.append(pltpu.async_copy(
                table_hbm.at[idx_v.at[pl.ds(j * ch, ch)]],
                rows_v.at[pl.ds(j * ch, ch)],
                sem,
            ))
        for c in cps:
            c.wait()

        nvec = d // 16

        def repack(g, _):
            row = g * 4
            woff = g * 4 * d
            for q in range(4):
                for h in range(nvec):
                    flat_v[pl.ds(woff + q * d + h * 16, 16)] = (
                        rows_v[row + q, pl.ds(h * 16, 16)]
                    )
            return 0

        lax.fori_loop(0, n_per_w // 4, repack, 0)
        pltpu.sync_copy(flat_v, out_hbm.at[pl.ds(wid * w_words, w_words)])

    return k(table, idx_flat)


def _attn_body(emb_ref, pos_ref, wk_ref, wq_ref, wv_ref, mask_ref, out_ref,
               *, scale, tp, d):
    e4 = emb_ref[...] + pos_ref[...]
    rp = e4.shape[0]
    e = jnp.concatenate(
        [lax.slice(e4, (0, j * d), (rp, (j + 1) * d)) for j in range(_PACK)],
        axis=0,
    )
    q = jnp.dot(e, wq_ref[...], preferred_element_type=jnp.float32)
    k = jnp.dot(e, wk_ref[...], preferred_element_type=jnp.float32)
    v = jnp.dot(e, wv_ref[...], preferred_element_type=jnp.float32)
    wei = lax.dot_general(q, k, (((1,), (1,)), ((), ())),
                          preferred_element_type=jnp.float32)
    wei = wei * scale + mask_ref[...]
    m = jnp.max(wei, axis=1, keepdims=True)
    p = jnp.exp(wei - m)
    s = jnp.sum(p, axis=1, keepdims=True)
    o = jnp.dot(p, v, preferred_element_type=jnp.float32) / s
    halves = []
    for rq in range(2):
        halves.append(jnp.concatenate(
            [lax.slice(o, ((2 * j + rq) * tp, 0), ((2 * j + rq + 1) * tp, d))
             for j in range(_PACK)],
            axis=1,
        ))
    out_ref[...] = jnp.concatenate(halves, axis=0)


def _tc_attn(emb_packed, pos_packed, Wk, Wq, Wv, mask_add, bb, tp, n_batch):
    D = Wk.shape[0]
    R = bb * tp
    RP = R // _PACK
    grid = n_batch // bb
    scale = float(D) ** -0.5
    return pl.pallas_call(
        functools.partial(_attn_body, scale=scale, tp=tp, d=D),
        grid=(grid,),
        in_specs=[
            pl.BlockSpec((RP, _PACK * D), lambda i: (i, 0)),
            pl.BlockSpec((RP, _PACK * D), lambda i: (0, 0)),
            pl.BlockSpec((D, D), lambda i: (0, 0)),
            pl.BlockSpec((D, D), lambda i: (0, 0)),
            pl.BlockSpec((D, D), lambda i: (0, 0)),
            pl.BlockSpec((R, R), lambda i: (0, 0)),
        ],
        out_specs=pl.BlockSpec((2 * tp, _PACK * D), lambda i: (i, 0)),
        out_shape=jax.ShapeDtypeStruct((n_batch // _PACK * tp, _PACK * D),
                                       jnp.float32),
        compiler_params=pltpu.CompilerParams(
            dimension_semantics=("parallel",),
        ),
    )(emb_packed, pos_packed, Wk, Wq, Wv, mask_add)


def _proj_body(o_ref, wlt_ref, bl_ref, out_ref, *, d):
    nb4 = o_ref.shape[0]
    ots = []
    for tl in range(_TCH):
        o4 = o_ref[:, tl, :]
        ots.append(jnp.concatenate(
            [lax.slice(o4, (0, j * d), (nb4, (j + 1) * d))
             for j in range(_PACK)],
            axis=0,
        ))
    ot = jnp.concatenate(ots, axis=0)
    lg = lax.dot_general(wlt_ref[...], ot, (((1,), (1,)), ((), ())),
                         preferred_element_type=jnp.float32)
    nb = nb4 * _PACK
    for tl in range(_TCH):
        out_ref[tl] = (lax.slice(lg, (0, tl * nb), (lg.shape[0], (tl + 1) * nb))
                       + bl_ref[...])


def _tc_proj(o3d, WlT, bl_col, t_out, tp):
    V, D = WlT.shape
    nb4 = o3d.shape[0]
    n_batch = nb4 * _PACK
    nbc = _BCH // _PACK
    grid = (pl.cdiv(t_out, _TCH), n_batch // _BCH)
    return pl.pallas_call(
        functools.partial(_proj_body, d=D),
        grid=grid,
        in_specs=[
            pl.BlockSpec((nbc, _TCH, _PACK * D), lambda u, bc: (bc, u, 0)),
            pl.BlockSpec((V, D), lambda u, bc: (0, 0)),
            pl.BlockSpec((V, 1), lambda u, bc: (0, 0)),
        ],
        out_specs=pl.BlockSpec((_TCH, V, _BCH), lambda u, bc: (u, 0, bc)),
        out_shape=jax.ShapeDtypeStruct((t_out, V, n_batch), jnp.float32),
        compiler_params=pltpu.CompilerParams(
            dimension_semantics=("parallel", "parallel"),
        ),
    )(o3d, WlT, bl_col)


def kernel(x, tok_table, pos_table, Wk, Wq, Wv, Wl, bl):
    B, T = x.shape
    V, D = tok_table.shape
    TP = 56
    N = B * TP

    BB = 8
    R = BB * TP
    RP = R // _PACK

    info = plsc.get_sparse_core_info()
    NW = info.num_cores * info.num_subcores
    n_per_w = N // NW
    CH = 112
    NCH = n_per_w // CH
    x_pad = jnp.pad(x.astype(jnp.int32), ((0, 0), (0, TP - T))).reshape(N)
    m = jnp.arange(N, dtype=jnp.int32)
    gi, mm = m // R, m % R
    tau = (mm % _PACK) * RP + mm // _PACK
    lv, tt = tau // TP, tau % TP
    b4 = 2 * gi + (lv % 2)
    nbc = _BCH // _PACK
    breal = (b4 // nbc) * _BCH + (lv // 2) * nbc + (b4 % nbc)
    idx_perm = x_pad[breal * TP + tt]
    emb_flat = _sc_gather(tok_table, idx_perm, NCH, CH)
    emb_packed = emb_flat.reshape(N // _PACK, _PACK * D)

    pos_pad = jnp.pad(pos_table, ((0, TP - T), (0, 0)))
    pos_packed = jnp.tile(pos_pad[jnp.arange(RP) % TP], (1, _PACK))
    r = jnp.arange(R)
    bidx, t = r // TP, r % TP
    causal = ((bidx[:, None] == bidx[None, :])
              & (t[:, None] >= t[None, :])
              & (t[None, :] < T))
    mask_add = jnp.where(causal, 0.0, -1e30).astype(jnp.float32)
    o2d = _tc_attn(emb_packed, pos_packed, Wk, Wq, Wv, mask_add, BB, TP, B)
    o3d = o2d.reshape(B // _PACK, TP, _PACK * D)

    out_tvb = _tc_proj(o3d, Wl.T, bl.reshape(V, 1), T, TP)
    return jnp.transpose(out_tvb, (2, 0, 1))


# speedup vs baseline: 2.2750x; 1.0703x over previous
import functools

import jax
import jax.numpy as jnp
from jax import lax
from jax.experimental import pallas as pl
from jax.experimental.pallas import tpu as pltpu
from jax.experimental.pallas import tpu_sc as plsc

_PACK = 4
_TCH = 8
_BCH = 128


def _sc_gather(table, idx_flat, nch, ch):
    n_total = idx_flat.shape[0]
    d = table.shape[1]
    n_per_w = nch * ch
    w_words = n_per_w * d
    mesh = plsc.VectorSubcoreMesh(core_axis_name="c", subcore_axis_name="s")
    info = plsc.get_sparse_core_info()
    nc = info.num_cores

    @functools.partial(
        pl.kernel,
        mesh=mesh,
        out_type=jax.ShapeDtypeStruct((n_total * d,), jnp.float32),
        scratch_types=[
            pltpu.VMEM((n_per_w,), jnp.int32),
            pltpu.VMEM((n_per_w, d), jnp.float32),
            pltpu.VMEM((w_words,), jnp.float32),
            pltpu.SemaphoreType.DMA,
        ],
        compiler_params=pltpu.CompilerParams(use_tc_tiling_on_sc=False),
    )
    def k(table_hbm, idx_hbm, out_hbm, idx_v, rows_v, flat_v, sem):
        wid = lax.axis_index("s") * nc + lax.axis_index("c")
        base = wid * n_per_w
        pltpu.sync_copy(idx_hbm.at[pl.ds(base, n_per_w)], idx_v)
        cps = []
        for j in range(nch):
            cps.append(pltpu.async_copy(
                table_hbm.at[idx_v.at[pl.ds(j * ch, ch)]],
                rows_v.at[pl.ds(j * ch, ch)],
                sem,
            ))
        for c in cps:
            c.wait()

        nvec = d // 16

        def repack(g, _):
            row = g * 4
            woff = g * 4 * d
            for q in range(4):
                for h in range(nvec):
                    flat_v[pl.ds(woff + q * d + h * 16, 16)] = (
                        rows_v[row + q, pl.ds(h * 16, 16)]
                    )
            return 0

        lax.fori_loop(0, n_per_w // 4, repack, 0)
        pltpu.sync_copy(flat_v, out_hbm.at[pl.ds(wid * w_words, w_words)])

    return k(table, idx_flat)


def _attn_body(emb_ref, pos_ref, wk_ref, wq_ref, wv_ref, mask_ref, out_ref,
               *, scale, tp, d):
    e4 = emb_ref[...] + pos_ref[...]
    rp = e4.shape[0]
    e = jnp.concatenate(
        [lax.slice(e4, (0, j * d), (rp, (j + 1) * d)) for j in range(_PACK)],
        axis=0,
    )
    bb = e.shape[0] // tp
    e3 = e.reshape(bb, tp, d)
    q3 = lax.dot_general(e3, wq_ref[...], (((2,), (0,)), ((), ())),
                         preferred_element_type=jnp.float32)
    k3 = lax.dot_general(e3, wk_ref[...], (((2,), (0,)), ((), ())),
                         preferred_element_type=jnp.float32)
    v3 = lax.dot_general(e3, wv_ref[...], (((2,), (0,)), ((), ())),
                         preferred_element_type=jnp.float32)
    wei = lax.dot_general(q3, k3, (((2,), (2,)), ((0,), (0,))),
                          preferred_element_type=jnp.float32)
    wei = wei * scale + mask_ref[...][None, :, :]
    m = jnp.max(wei, axis=2, keepdims=True)
    p = jnp.exp(wei - m)
    s = jnp.sum(p, axis=2, keepdims=True)
    o3 = lax.dot_general(p, v3, (((2,), (1,)), ((0,), (0,))),
                         preferred_element_type=jnp.float32) / s
    halves = []
    for rq in range(2):
        halves.append(jnp.concatenate(
            [o3[2 * j + rq] for j in range(_PACK)],
            axis=1,
        ))
    out_ref[...] = jnp.concatenate(halves, axis=0)


def _tc_attn(emb_packed, pos_packed, Wk, Wq, Wv, mask_add, bb, tp, n_batch):
    D = Wk.shape[0]
    R = bb * tp
    RP = R // _PACK
    grid = n_batch // bb
    scale = float(D) ** -0.5
    return pl.pallas_call(
        functools.partial(_attn_body, scale=scale, tp=tp, d=D),
        grid=(grid,),
        in_specs=[
            pl.BlockSpec((RP, _PACK * D), lambda i: (i, 0)),
            pl.BlockSpec((RP, _PACK * D), lambda i: (0, 0)),
            pl.BlockSpec((D, D), lambda i: (0, 0)),
            pl.BlockSpec((D, D), lambda i: (0, 0)),
            pl.BlockSpec((D, D), lambda i: (0, 0)),
            pl.BlockSpec((tp, tp), lambda i: (0, 0)),
        ],
        out_specs=pl.BlockSpec((2 * tp, _PACK * D), lambda i: (i, 0)),
        out_shape=jax.ShapeDtypeStruct((n_batch // _PACK * tp, _PACK * D),
                                       jnp.float32),
        compiler_params=pltpu.CompilerParams(
            dimension_semantics=("parallel",),
        ),
    )(emb_packed, pos_packed, Wk, Wq, Wv, mask_add)


def _proj_body(o_ref, wlt_ref, bl_ref, out_ref, *, d):
    nb4 = o_ref.shape[0]
    ots = []
    for tl in range(_TCH):
        o4 = o_ref[:, tl, :]
        ots.append(jnp.concatenate(
            [lax.slice(o4, (0, j * d), (nb4, (j + 1) * d))
             for j in range(_PACK)],
            axis=0,
        ))
    ot = jnp.concatenate(ots, axis=0)
    lg = lax.dot_general(wlt_ref[...], ot, (((1,), (1,)), ((), ())),
                         preferred_element_type=jnp.float32)
    nb = nb4 * _PACK
    for tl in range(_TCH):
        out_ref[tl] = (lax.slice(lg, (0, tl * nb), (lg.shape[0], (tl + 1) * nb))
                       + bl_ref[...])


def _tc_proj(o3d, WlT, bl_col, t_out, tp):
    V, D = WlT.shape
    nb4 = o3d.shape[0]
    n_batch = nb4 * _PACK
    nbc = _BCH // _PACK
    grid = (pl.cdiv(t_out, _TCH), n_batch // _BCH)
    return pl.pallas_call(
        functools.partial(_proj_body, d=D),
        grid=grid,
        in_specs=[
            pl.BlockSpec((nbc, _TCH, _PACK * D), lambda u, bc: (bc, u, 0)),
            pl.BlockSpec((V, D), lambda u, bc: (0, 0)),
            pl.BlockSpec((V, 1), lambda u, bc: (0, 0)),
        ],
        out_specs=pl.BlockSpec((_TCH, V, _BCH), lambda u, bc: (u, 0, bc)),
        out_shape=jax.ShapeDtypeStruct((t_out, V, n_batch), jnp.float32),
        compiler_params=pltpu.CompilerParams(
            dimension_semantics=("parallel", "parallel"),
        ),
    )(o3d, WlT, bl_col)


def kernel(x, tok_table, pos_table, Wk, Wq, Wv, Wl, bl):
    B, T = x.shape
    V, D = tok_table.shape
    TP = 56
    N = B * TP

    BB = 8
    R = BB * TP
    RP = R // _PACK

    info = plsc.get_sparse_core_info()
    NW = info.num_cores * info.num_subcores
    n_per_w = N // NW
    CH = 112
    NCH = n_per_w // CH
    x_pad = jnp.pad(x.astype(jnp.int32), ((0, 0), (0, TP - T))).reshape(N)
    m = jnp.arange(N, dtype=jnp.int32)
    gi, mm = m // R, m % R
    tau = (mm % _PACK) * RP + mm // _PACK
    lv, tt = tau // TP, tau % TP
    b4 = 2 * gi + (lv % 2)
    nbc = _BCH // _PACK
    breal = (b4 // nbc) * _BCH + (lv // 2) * nbc + (b4 % nbc)
    idx_perm = x_pad[breal * TP + tt]
    emb_flat = _sc_gather(tok_table, idx_perm, NCH, CH)
    emb_packed = emb_flat.reshape(N // _PACK, _PACK * D)

    pos_pad = jnp.pad(pos_table, ((0, TP - T), (0, 0)))
    pos_packed = jnp.tile(pos_pad[jnp.arange(RP) % TP], (1, _PACK))
    t = jnp.arange(TP)
    causal = (t[:, None] >= t[None, :]) & (t[None, :] < T)
    mask_add = jnp.where(causal, 0.0, -1e30).astype(jnp.float32)
    o2d = _tc_attn(emb_packed, pos_packed, Wk, Wq, Wv, mask_add, BB, TP, B)
    o3d = o2d.reshape(B // _PACK, TP, _PACK * D)

    out_tvb = _tc_proj(o3d, Wl.T, bl.reshape(V, 1), T, TP)
    return jnp.transpose(out_tvb, (2, 0, 1))


# speedup vs baseline: 2.7576x; 1.2122x over previous
import functools

import jax
import jax.numpy as jnp
from jax import lax
from jax.experimental import pallas as pl
from jax.experimental.pallas import tpu as pltpu
from jax.experimental.pallas import tpu_sc as plsc

_PACK = 4
_TCH = 8
_BCH = 128


def _sc_gather(table, idx_flat, nch, ch):
    n_total = idx_flat.shape[0]
    d = table.shape[1]
    n_per_w = nch * ch
    w_words = n_per_w * d
    mesh = plsc.VectorSubcoreMesh(core_axis_name="c", subcore_axis_name="s")
    info = plsc.get_sparse_core_info()
    nc = info.num_cores

    @functools.partial(
        pl.kernel,
        mesh=mesh,
        out_type=jax.ShapeDtypeStruct((n_total * d,), jnp.float32),
        scratch_types=[
            pltpu.VMEM((n_per_w,), jnp.int32),
            pltpu.VMEM((n_per_w, d), jnp.float32),
            pltpu.VMEM((w_words,), jnp.float32),
            pltpu.SemaphoreType.DMA,
        ],
        compiler_params=pltpu.CompilerParams(use_tc_tiling_on_sc=False),
    )
    def k(table_hbm, idx_hbm, out_hbm, idx_v, rows_v, flat_v, sem):
        wid = lax.axis_index("s") * nc + lax.axis_index("c")
        base = wid * n_per_w
        pltpu.sync_copy(idx_hbm.at[pl.ds(base, n_per_w)], idx_v)
        cps = []
        for j in range(nch):
            cps.append(pltpu.async_copy(
                table_hbm.at[idx_v.at[pl.ds(j * ch, ch)]],
                rows_v.at[pl.ds(j * ch, ch)],
                sem,
            ))
        for c in cps:
            c.wait()

        nvec = d // 16

        def repack(g, _):
            row = g * 4
            woff = g * 4 * d
            for q in range(4):
                for h in range(nvec):
                    flat_v[pl.ds(woff + q * d + h * 16, 16)] = (
                        rows_v[row + q, pl.ds(h * 16, 16)]
                    )
            return 0

        lax.fori_loop(0, n_per_w // 4, repack, 0)
        pltpu.sync_copy(flat_v, out_hbm.at[pl.ds(wid * w_words, w_words)])

    return k(table, idx_flat)


def _attn_body(emb_ref, pos_ref, wk_ref, wq_ref, wv_ref, mask_ref, out_ref,
               *, scale, tp, d):
    e4 = emb_ref[...] + pos_ref[...]
    rp = e4.shape[0]
    e = jnp.concatenate(
        [lax.slice(e4, (0, j * d), (rp, (j + 1) * d)) for j in range(_PACK)],
        axis=0,
    )
    bb = e.shape[0] // tp
    e3 = e.reshape(bb, tp, d)
    q3 = lax.dot_general(e3, wq_ref[...], (((2,), (0,)), ((), ())),
                         preferred_element_type=jnp.float32)
    k3 = lax.dot_general(e3, wk_ref[...], (((2,), (0,)), ((), ())),
                         preferred_element_type=jnp.float32)
    v3 = lax.dot_general(e3, wv_ref[...], (((2,), (0,)), ((), ())),
                         preferred_element_type=jnp.float32)
    wei = lax.dot_general(q3, k3, (((2,), (2,)), ((0,), (0,))),
                          preferred_element_type=jnp.float32)
    wei = wei * scale + mask_ref[...][None, :, :]
    m = jnp.max(wei, axis=2, keepdims=True)
    p = jnp.exp(wei - m)
    s = jnp.sum(p, axis=2, keepdims=True)
    o3 = lax.dot_general(p, v3, (((2,), (1,)), ((0,), (0,))),
                         preferred_element_type=jnp.float32) / s
    nb4blk = bb // _PACK
    halves = []
    for rq in range(nb4blk):
        halves.append(jnp.concatenate(
            [o3[nb4blk * j + rq] for j in range(_PACK)],
            axis=1,
        ))
    out_ref[...] = jnp.concatenate(halves, axis=0)


def _tc_attn(emb_packed, pos_packed, Wk, Wq, Wv, mask_add, bb, tp, n_batch):
    D = Wk.shape[0]
    R = bb * tp
    RP = R // _PACK
    grid = n_batch // bb
    scale = float(D) ** -0.5
    return pl.pallas_call(
        functools.partial(_attn_body, scale=scale, tp=tp, d=D),
        grid=(grid,),
        in_specs=[
            pl.BlockSpec((RP, _PACK * D), lambda i: (i, 0)),
            pl.BlockSpec((RP, _PACK * D), lambda i: (0, 0)),
            pl.BlockSpec((D, D), lambda i: (0, 0)),
            pl.BlockSpec((D, D), lambda i: (0, 0)),
            pl.BlockSpec((D, D), lambda i: (0, 0)),
            pl.BlockSpec((tp, tp), lambda i: (0, 0)),
        ],
        out_specs=pl.BlockSpec((bb // _PACK * tp, _PACK * D), lambda i: (i, 0)),
        out_shape=jax.ShapeDtypeStruct((n_batch // _PACK * tp, _PACK * D),
                                       jnp.float32),
        compiler_params=pltpu.CompilerParams(
            dimension_semantics=("parallel",),
        ),
    )(emb_packed, pos_packed, Wk, Wq, Wv, mask_add)


def _proj_body(o_ref, wlt_ref, bl_ref, out_ref, *, d):
    nb4 = o_ref.shape[0]
    ots = []
    for tl in range(_TCH):
        o4 = o_ref[:, tl, :]
        ots.append(jnp.concatenate(
            [lax.slice(o4, (0, j * d), (nb4, (j + 1) * d))
             for j in range(_PACK)],
            axis=0,
        ))
    ot = jnp.concatenate(ots, axis=0)
    lg = lax.dot_general(wlt_ref[...], ot, (((1,), (1,)), ((), ())),
                         preferred_element_type=jnp.float32)
    nb = nb4 * _PACK
    for tl in range(_TCH):
        out_ref[tl] = (lax.slice(lg, (0, tl * nb), (lg.shape[0], (tl + 1) * nb))
                       + bl_ref[...])


def _tc_proj(o3d, WlT, bl_col, t_out, tp):
    V, D = WlT.shape
    nb4 = o3d.shape[0]
    n_batch = nb4 * _PACK
    nbc = _BCH // _PACK
    grid = (pl.cdiv(t_out, _TCH), n_batch // _BCH)
    return pl.pallas_call(
        functools.partial(_proj_body, d=D),
        grid=grid,
        in_specs=[
            pl.BlockSpec((nbc, _TCH, _PACK * D), lambda u, bc: (bc, u, 0)),
            pl.BlockSpec((V, D), lambda u, bc: (0, 0)),
            pl.BlockSpec((V, 1), lambda u, bc: (0, 0)),
        ],
        out_specs=pl.BlockSpec((_TCH, V, _BCH), lambda u, bc: (u, 0, bc)),
        out_shape=jax.ShapeDtypeStruct((t_out, V, n_batch), jnp.float32),
        compiler_params=pltpu.CompilerParams(
            dimension_semantics=("parallel", "parallel"),
        ),
    )(o3d, WlT, bl_col)


def kernel(x, tok_table, pos_table, Wk, Wq, Wv, Wl, bl):
    B, T = x.shape
    V, D = tok_table.shape
    TP = 56
    N = B * TP

    BB = 32
    R = BB * TP
    RP = R // _PACK

    info = plsc.get_sparse_core_info()
    NW = info.num_cores * info.num_subcores
    n_per_w = N // NW
    CH = 112
    NCH = n_per_w // CH
    x_pad = jnp.pad(x.astype(jnp.int32), ((0, 0), (0, TP - T))).reshape(N)
    m = jnp.arange(N, dtype=jnp.int32)
    gi, mm = m // R, m % R
    tau = (mm % _PACK) * RP + mm // _PACK
    lv, tt = tau // TP, tau % TP
    nb4blk = BB // _PACK
    b4 = nb4blk * gi + (lv % nb4blk)
    jslot = lv // nb4blk
    nbc = _BCH // _PACK
    breal = (b4 // nbc) * _BCH + jslot * nbc + (b4 % nbc)
    idx_perm = x_pad[breal * TP + tt]
    emb_flat = _sc_gather(tok_table, idx_perm, NCH, CH)
    emb_packed = emb_flat.reshape(N // _PACK, _PACK * D)

    pos_pad = jnp.pad(pos_table, ((0, TP - T), (0, 0)))
    pos_packed = jnp.tile(pos_pad[jnp.arange(RP) % TP], (1, _PACK))
    t = jnp.arange(TP)
    causal = (t[:, None] >= t[None, :]) & (t[None, :] < T)
    mask_add = jnp.where(causal, 0.0, -1e30).astype(jnp.float32)
    o2d = _tc_attn(emb_packed, pos_packed, Wk, Wq, Wv, mask_add, BB, TP, B)
    o3d = o2d.reshape(B // _PACK, TP, _PACK * D)

    out_tvb = _tc_proj(o3d, Wl.T, bl.reshape(V, 1), T, TP)
    return jnp.transpose(out_tvb, (2, 0, 1))


# speedup vs baseline: 2.8967x; 1.0504x over previous
import functools

import jax
import jax.numpy as jnp
from jax import lax
from jax.experimental import pallas as pl
from jax.experimental.pallas import tpu as pltpu
from jax.experimental.pallas import tpu_sc as plsc

_PACK = 4
_TCH = 8
_BCH = 128


def _sc_gather(table, idx_flat, nch, ch):
    n_total = idx_flat.shape[0]
    d = table.shape[1]
    n_per_w = nch * ch
    w_words = n_per_w * d
    mesh = plsc.VectorSubcoreMesh(core_axis_name="c", subcore_axis_name="s")
    info = plsc.get_sparse_core_info()
    nc = info.num_cores

    @functools.partial(
        pl.kernel,
        mesh=mesh,
        out_type=jax.ShapeDtypeStruct((n_total * d,), jnp.float32),
        scratch_types=[
            pltpu.VMEM((n_per_w,), jnp.int32),
            pltpu.VMEM((n_per_w, d), jnp.float32),
            pltpu.VMEM((w_words,), jnp.float32),
            pltpu.SemaphoreType.DMA,
        ],
        compiler_params=pltpu.CompilerParams(use_tc_tiling_on_sc=False),
    )
    def k(table_hbm, idx_hbm, out_hbm, idx_v, rows_v, flat_v, sem):
        wid = lax.axis_index("s") * nc + lax.axis_index("c")
        base = wid * n_per_w
        pltpu.sync_copy(idx_hbm.at[pl.ds(base, n_per_w)], idx_v)
        cps = []
        for j in range(nch):
            cps.append(pltpu.async_copy(
                table_hbm.at[idx_v.at[pl.ds(j * ch, ch)]],
                rows_v.at[pl.ds(j * ch, ch)],
                sem,
            ))
        for c in cps:
            c.wait()

        nvec = d // 16

        def repack(g, _):
            row = g * 4
            woff = g * 4 * d
            for q in range(4):
                for h in range(nvec):
                    flat_v[pl.ds(woff + q * d + h * 16, 16)] = (
                        rows_v[row + q, pl.ds(h * 16, 16)]
                    )
            return 0

        lax.fori_loop(0, n_per_w // 4, repack, 0)
        pltpu.sync_copy(flat_v, out_hbm.at[pl.ds(wid * w_words, w_words)])

    return k(table, idx_flat)


def _attn_body(emb_ref, pos_ref, wk_ref, wq_ref, wv_ref, mask_ref, out_ref,
               *, scale, tp, d):
    e4 = emb_ref[...] + pos_ref[...]
    rp = e4.shape[0]
    e = jnp.concatenate(
        [lax.slice(e4, (0, j * d), (rp, (j + 1) * d)) for j in range(_PACK)],
        axis=0,
    )
    bb = e.shape[0] // tp
    e3 = e.reshape(bb, tp, d)
    q3 = lax.dot_general(e3, wq_ref[...], (((2,), (0,)), ((), ())),
                         preferred_element_type=jnp.float32)
    k3 = lax.dot_general(e3, wk_ref[...], (((2,), (0,)), ((), ())),
                         preferred_element_type=jnp.float32)
    v3 = lax.dot_general(e3, wv_ref[...], (((2,), (0,)), ((), ())),
                         preferred_element_type=jnp.float32)
    wei = lax.dot_general(q3, k3, (((2,), (2,)), ((0,), (0,))),
                          preferred_element_type=jnp.float32)
    wei = wei * scale + mask_ref[...][None, :, :]
    m = jnp.max(wei, axis=2, keepdims=True)
    p = jnp.exp(wei - m)
    s = jnp.sum(p, axis=2, keepdims=True)
    o3 = lax.dot_general(p, v3, (((2,), (1,)), ((0,), (0,))),
                         preferred_element_type=jnp.float32) / s
    nb4blk = bb // _PACK
    halves = []
    for rq in range(nb4blk):
        halves.append(jnp.concatenate(
            [o3[nb4blk * j + rq] for j in range(_PACK)],
            axis=1,
        ))
    out_ref[...] = jnp.concatenate(halves, axis=0)


def _tc_attn(emb_packed, pos_packed, Wk, Wq, Wv, mask_add, bb, tp, n_batch):
    D = Wk.shape[0]
    R = bb * tp
    RP = R // _PACK
    grid = n_batch // bb
    scale = float(D) ** -0.5
    return pl.pallas_call(
        functools.partial(_attn_body, scale=scale, tp=tp, d=D),
        grid=(grid,),
        in_specs=[
            pl.BlockSpec((RP, _PACK * D), lambda i: (i, 0)),
            pl.BlockSpec((RP, _PACK * D), lambda i: (0, 0)),
            pl.BlockSpec((D, D), lambda i: (0, 0)),
            pl.BlockSpec((D, D), lambda i: (0, 0)),
            pl.BlockSpec((D, D), lambda i: (0, 0)),
            pl.BlockSpec((tp, tp), lambda i: (0, 0)),
        ],
        out_specs=pl.BlockSpec((bb // _PACK * tp, _PACK * D), lambda i: (i, 0)),
        out_shape=jax.ShapeDtypeStruct((n_batch // _PACK * tp, _PACK * D),
                                       jnp.float32),
        compiler_params=pltpu.CompilerParams(
            dimension_semantics=("parallel",),
        ),
    )(emb_packed, pos_packed, Wk, Wq, Wv, mask_add)


def _proj_body(o_ref, wlt_ref, bl_ref, out_ref, *, d):
    nb4 = o_ref.shape[0]
    ots = []
    for tl in range(_TCH):
        o4 = o_ref[:, tl, :]
        ots.append(jnp.concatenate(
            [lax.slice(o4, (0, j * d), (nb4, (j + 1) * d))
             for j in range(_PACK)],
            axis=0,
        ))
    ot = jnp.concatenate(ots, axis=0)
    lg = lax.dot_general(wlt_ref[...], ot, (((1,), (1,)), ((), ())),
                         preferred_element_type=jnp.float32)
    nb = nb4 * _PACK
    for tl in range(_TCH):
        out_ref[tl] = (lax.slice(lg, (0, tl * nb), (lg.shape[0], (tl + 1) * nb))
                       + bl_ref[...])


def _tc_proj(o3d, WlT, bl_col, t_out, tp):
    V, D = WlT.shape
    nb4 = o3d.shape[0]
    n_batch = nb4 * _PACK
    nbc = _BCH // _PACK
    grid = (pl.cdiv(t_out, _TCH), n_batch // _BCH)
    return pl.pallas_call(
        functools.partial(_proj_body, d=D),
        grid=grid,
        in_specs=[
            pl.BlockSpec((nbc, _TCH, _PACK * D), lambda u, bc: (bc, u, 0)),
            pl.BlockSpec((V, D), lambda u, bc: (0, 0)),
            pl.BlockSpec((V, 1), lambda u, bc: (0, 0)),
        ],
        out_specs=pl.BlockSpec((_TCH, V, _BCH), lambda u, bc: (u, 0, bc)),
        out_shape=jax.ShapeDtypeStruct((t_out, V, n_batch), jnp.float32),
        compiler_params=pltpu.CompilerParams(
            dimension_semantics=("parallel", "parallel"),
        ),
    )(o3d, WlT, bl_col)


def kernel(x, tok_table, pos_table, Wk, Wq, Wv, Wl, bl):
    B, T = x.shape
    V, D = tok_table.shape
    TP = 56
    N = B * TP

    BB = 64
    R = BB * TP
    RP = R // _PACK

    info = plsc.get_sparse_core_info()
    NW = info.num_cores * info.num_subcores
    n_per_w = N // NW
    CH = 112
    NCH = n_per_w // CH
    x_pad = jnp.pad(x.astype(jnp.int32), ((0, 0), (0, TP - T))).reshape(N)
    m = jnp.arange(N, dtype=jnp.int32)
    gi, mm = m // R, m % R
    tau = (mm % _PACK) * RP + mm // _PACK
    lv, tt = tau // TP, tau % TP
    nb4blk = BB // _PACK
    b4 = nb4blk * gi + (lv % nb4blk)
    jslot = lv // nb4blk
    nbc = _BCH // _PACK
    breal = (b4 // nbc) * _BCH + jslot * nbc + (b4 % nbc)
    idx_perm = x_pad[breal * TP + tt]
    emb_flat = _sc_gather(tok_table, idx_perm, NCH, CH)
    emb_packed = emb_flat.reshape(N // _PACK, _PACK * D)

    pos_pad = jnp.pad(pos_table, ((0, TP - T), (0, 0)))
    pos_packed = jnp.tile(pos_pad[jnp.arange(RP) % TP], (1, _PACK))
    t = jnp.arange(TP)
    causal = (t[:, None] >= t[None, :]) & (t[None, :] < T)
    mask_add = jnp.where(causal, 0.0, -1e30).astype(jnp.float32)
    o2d = _tc_attn(emb_packed, pos_packed, Wk, Wq, Wv, mask_add, BB, TP, B)
    o3d = o2d.reshape(B // _PACK, TP, _PACK * D)

    out_tvb = _tc_proj(o3d, Wl.T, bl.reshape(V, 1), T, TP)
    return jnp.transpose(out_tvb, (2, 0, 1))
